# Initial kernel scaffold; baseline (speedup 1.0000x reference)
import jax
import jax.numpy as jnp
from jax.experimental import pallas as pl


def kernel(xCellCenters, xFace, params, edge_attrs, edge_indices):
    raise NotImplementedError("write your pallas kernel here")


# speedup vs baseline: 2.5093x; 2.5093x over previous
import functools

import jax
import jax.numpy as jnp
from jax import lax
from jax.experimental import pallas as pl
from jax.experimental.pallas import tpu as pltpu
from jax.experimental.pallas import tpu_sc as plsc

F32 = jnp.float32
I32 = jnp.int32

NCORES = 2
NSUB = 16
NTILES = NCORES * NSUB
CHS = 512
SUB = 128
EUNIT = NTILES * CHS
EPS = 1e-5


def _ceil_to(x, m):
    return (x + m - 1) // m * m


def _lane_take(v, idx):
    return lax.gather(
        v, idx[:, None],
        dimension_numbers=lax.GatherDimensionNumbers(
            offset_dims=(), collapsed_slice_dims=(0,), start_index_map=(0,)),
        slice_sizes=(1,), mode=lax.GatherScatterMode.PROMISE_IN_BOUNDS)


@functools.lru_cache(maxsize=None)
def _make_edge_agg(n_src, n_rows, k, e_pad, pack2):
    n_rows_pad = _ceil_to(n_rows, 512)
    rpt = n_rows_pad // NSUB
    ept = e_pad // NTILES
    e_half = e_pad // NCORES
    chunks = ept // CHS

    mesh = plsc.VectorSubcoreMesh(
        core_axis_name="c", subcore_axis_name="s",
        num_cores=NCORES, num_subcores=NSUB)

    def body(tbl, src, dst, ea, zeros_hbm, out, acc, srcb, dstb, eab, dloc,
             rows, sem, sem2):
        c = lax.axis_index("c")
        s = lax.axis_index("s")
        pltpu.sync_copy(zeros_hbm.at[pl.ds(s * rpt, rpt)],
                        acc.at[pl.ds(s * rpt, rpt)])
        plsc.subcore_barrier()

        base = c * e_half + s * ept
        lanes = lax.broadcasted_iota(I32, (16,), 0)

        def chunk_body(ch, carry):
            eb = base + ch * CHS
            c1 = pltpu.async_copy(src.at[pl.ds(eb, CHS)], srcb, sem)
            c2 = pltpu.async_copy(dst.at[pl.ds(eb, CHS)], dstb, sem)
            c3 = pltpu.async_copy(ea.at[pl.ds(eb, CHS)], eab, sem)
            c1.wait(); c2.wait(); c3.wait()
            for sub in range(CHS // SUB):
                g = pltpu.async_copy(
                    tbl.at[srcb.at[pl.ds(sub * SUB, SUB)]], rows, sem2)
                g.wait()
                for grp in range(SUB // 16):
                    off = sub * SUB + grp * 16
                    ea_v = eab[pl.ds(off, 16)]
                    dstv = dstb[pl.ds(off, 16)]
                    if pack2:
                        dloc[pl.ds(grp * 16, 16)] = \
                            lax.shift_right_logical(dstv, 1)
                        par = dstv & 1
                    else:
                        dloc[pl.ds(grp * 16, 16)] = dstv
                    for e in range(16):
                        le = grp * 16 + e
                        idx_e = jnp.full((16,), e, I32)
                        scale = _lane_take(ea_v, idx_e)
                        if pack2:
                            pb = _lane_take(par, idx_e)
                            ridx = (lanes + lax.shift_left(pb, 3)) & 15
                        for h in range(k):
                            v = rows[le, h] * scale
                            if pack2:
                                v = _lane_take(v, ridx)
                            rows[le, h] = v
                pltpu.sync_copy(rows, acc.at[dloc], add=True)
            return carry

        lax.fori_loop(0, chunks, chunk_body, 0)
        plsc.subcore_barrier()
        pltpu.sync_copy(acc.at[pl.ds(s * rpt, rpt)],
                        out.at[c, pl.ds(s * rpt, rpt)])

    return pl.kernel(
        body,
        out_type=jax.ShapeDtypeStruct((NCORES, n_rows_pad, k, 16), F32),
        mesh=mesh,
        scratch_types=[
            pltpu.VMEM_SHARED((n_rows_pad, k, 16), F32),
            pltpu.VMEM((CHS,), I32),
            pltpu.VMEM((CHS,), I32),
            pltpu.VMEM((CHS,), F32),
            pltpu.VMEM((SUB,), I32),
            pltpu.VMEM((SUB, k, 16), F32),
            pltpu.SemaphoreType.DMA,
            pltpu.SemaphoreType.DMA,
        ],
        compiler_params=pltpu.CompilerParams(use_tc_tiling_on_sc=False),
    )


def _edge_agg(table, ei, ea, n_dst, pack2=False):
    nr, c = table.shape
    k = c // 16
    e = ei.shape[1]
    e_pad = _ceil_to(e, EUNIT)
    src = ei[0]
    dst = ei[1]
    if e_pad != e:
        pad = e_pad - e
        src = jnp.concatenate([src, jnp.zeros((pad,), I32)])
        dst = jnp.concatenate([dst, jnp.zeros((pad,), I32)])
        ea = jnp.concatenate([ea, jnp.zeros((pad,), F32)])
    n_rows = n_dst // 2 if pack2 else n_dst
    n_rows_pad = _ceil_to(n_rows, 512)
    zeros_hbm = jnp.zeros((n_rows_pad, k, 16), F32)
    fn = _make_edge_agg(nr, n_rows, k, e_pad, pack2)
    out = fn(table.reshape(nr, k, 16), src, dst, ea, zeros_hbm)
    if pack2:
        return (out[0].reshape(n_rows_pad * 2, 8),
                out[1].reshape(n_rows_pad * 2, 8))
    return (out[0].reshape(n_rows_pad, c), out[1].reshape(n_rows_pad, c))


def _dense(n, co, terms, bias=None, relu=False, stats=False, want_out=True):
    np_ = _ceil_to(n, 8)
    r = 1000 if np_ % 1000 == 0 else np_
    grid = np_ // r
    has_pad_rows = np_ != n

    cfg = []
    operands = []
    in_specs = []

    def add_full(a, ci):
        operands.append(a)
        in_specs.append(pl.BlockSpec((1, ci), lambda i: (0, 0)))

    for (xs, tb, trelu, norm, w) in terms:
        ci = xs[0].shape[1]
        for x in xs:
            operands.append(x)
            in_specs.append(pl.BlockSpec((r, ci), lambda i: (i, 0)))
        if tb is not None:
            add_full(tb, ci)
        cnt = None
        if norm is not None:
            ssum, ssq, cnt = norm
            add_full(ssum, ci)
            add_full(ssq, ci)
        if w is not None:
            operands.append(w)
            in_specs.append(pl.BlockSpec(w.shape, lambda i: (0, 0)))
        cfg.append((len(xs), tb is not None, trelu, norm is not None,
                    w is not None, float(cnt) if cnt else 0.0))
    if bias is not None:
        add_full(bias, co)

    out_shapes = []
    out_specs = []
    if want_out:
        out_shapes.append(jax.ShapeDtypeStruct((np_, co), F32))
        out_specs.append(pl.BlockSpec((r, co), lambda i: (i, 0)))
    if stats:
        for _ in range(2):
            out_shapes.append(jax.ShapeDtypeStruct((1, co), F32))
            out_specs.append(pl.BlockSpec((1, co), lambda i: (0, 0)))

    n_terms = len(cfg)
    has_bias = bias is not None

    def body(*refs):
        it = iter(refs)
        acc = None
        for (nxs, has_tb, trelu, has_norm, has_w, cnt) in cfg:
            xb = next(it)[...]
            for _ in range(nxs - 1):
                xb = xb + next(it)[...]
            if has_tb:
                xb = xb + next(it)[...]
            if trelu:
                xb = jnp.maximum(xb, 0.0)
            if has_norm:
                m = next(it)[...] / cnt
                v = next(it)[...] / cnt - m * m
                xb = (xb - m) * lax.rsqrt(v + EPS)
            if has_w:
                xb = jnp.dot(xb, next(it)[...], preferred_element_type=F32)
            acc = xb if acc is None else acc + xb
        if has_bias:
            acc = acc + next(it)[...]
        if relu:
            acc = jnp.maximum(acc, 0.0)
        if want_out:
            next(it)[...] = acc
        if stats:
            sr = next(it)
            qr = next(it)
            i = pl.program_id(0)
            a = acc
            if has_pad_rows:
                row = lax.broadcasted_iota(I32, (r, 1), 0) + i * r
                a = jnp.where(row < n, acc, 0.0)
            bs = jnp.sum(a, axis=0, keepdims=True)
            bq = jnp.sum(a * a, axis=0, keepdims=True)

            @pl.when(i == 0)
            def _():
                sr[...] = bs
                qr[...] = bq

            @pl.when(i > 0)
            def _():
                sr[...] = sr[...] + bs
                qr[...] = qr[...] + bq

    res = pl.pallas_call(
        body,
        grid=(grid,),
        in_specs=in_specs,
        out_specs=out_specs if len(out_specs) > 1 else out_specs[0],
        out_shape=out_shapes if len(out_shapes) > 1 else out_shapes[0],
    )(*operands)
    return res


def _pad2(w, ri, co):
    out = jnp.zeros((ri, co), F32)
    return out.at[:w.shape[0], :w.shape[1]].set(w)


def _padb(b, co):
    return jnp.zeros((1, co), F32).at[0, :b.shape[0]].set(b)


def _gconv_pp_pallas(x_terms, b, ei, ea, n, co_pad, stats=False):
    wn_terms = [(xs, None, False, None, wn_i) for (xs, _, wn_i) in x_terms]
    wr_terms = [(xs, None, False, None, wr_i) for (xs, wr_i, _) in x_terms]
    xwn = _dense(n, co_pad, wn_terms)
    a0, a1 = _edge_agg(xwn, ei, ea, n)
    out_terms = wr_terms + [([a0, a1], None, False, None, None)]
    return _dense(n, co_pad, out_terms, bias=_padb(b, co_pad), relu=True,
                  stats=stats)


def kernel(xCellCenters, xFace, params, edge_attrs, edge_indices):
    p = params
    ea = edge_attrs
    ei = edge_indices
    nc = xCellCenters.shape[1]
    nf = xFace.shape[1]
    n0 = 100000
    n1 = 25000
    n2 = 6250
    n3 = 1600

    xcc = xCellCenters[0]
    xf = xFace[0]

    scc, qcc = _dense(nc, 2, [([xcc], None, False, None, None)],
                      stats=True, want_out=False)
    sf, qf = _dense(nf, 4, [([xf], None, False, None, None)],
                    stats=True, want_out=False)

    c1 = p['convN_1']
    xwcf = _dense(nc, 16, [([xcc], None, False, (scc, qcc, nc),
                            _pad2(c1['W_cf'], 2, 16))])
    acf0, acf1 = _edge_agg(xwcf, ei['cf'], ea['cf'], nf, pack2=True)

    wfp = c1['W_fp']
    xwfp = _dense(nf, 16, [
        ([acf0, acf1], _padb(c1['b_cf'], 8), True, None, _pad2(wfp[:8], 8, 16)),
        ([xf], None, False, (sf, qf, nf), _pad2(wfp[8:], 4, 16)),
    ])
    afp0, afp1 = _edge_agg(xwfp, ei['fp'], ea['fp'], n0)
    h0 = _dense(n0, 16, [([afp0, afp1], _padb(c1['b_fp'], 16), True,
                          None, None)])

    x1, s1, q1 = _gconv_pp_pallas(
        [([h0], _pad2(c1['Wr_pp'], 16, 16), _pad2(c1['Wn_pp'], 16, 16))],
        c1['b_pp'], ei['pp0'], ea['pp0'], n0, 16, stats=True)
    x1n = _dense(n0, 16, [([x1], None, False, (s1, q1, n0), None)])

    def _ppt(xs, lw, ci_pad, co_pad):
        return (xs, _pad2(lw['Wr'], ci_pad, co_pad),
                _pad2(lw['Wn'], ci_pad, co_pad))

    ap0, ap1 = _edge_agg(x1n, ei['pool01'], ea['pool01'], n1)
    c2 = p['convN_2']
    h = _gconv_pp_pallas([_ppt([ap0, ap1], c2['l1'], 16, 16)], c2['l1']['b'],
                         ei['pp1'], ea['pp1'], n1, 16)
    x2, s2, q2 = _gconv_pp_pallas([_ppt([h], c2['l2'], 16, 16)], c2['l2']['b'],
                                  ei['pp1'], ea['pp1'], n1, 16, stats=True)
    x2n = _dense(n1, 16, [([x2], None, False, (s2, q2, n1), None)])

    bp0, bp1 = _edge_agg(x2n, ei['pool12'], ea['pool12'], n2)
    c3 = p['convN_3']
    h = _gconv_pp_pallas([_ppt([bp0, bp1], c3['l1'], 16, 16)], c3['l1']['b'],
                         ei['pp2'], ea['pp2'], n2, 16)
    x3, s3, q3 = _gconv_pp_pallas([_ppt([h], c3['l2'], 16, 16)], c3['l2']['b'],
                                  ei['pp2'], ea['pp2'], n2, 16, stats=True)
    x3n = _dense(n2, 16, [([x3], None, False, (s3, q3, n2), None)])

    cp0, cp1 = _edge_agg(x3n, ei['pool23'], ea['pool23'], n3)
    c4 = p['convN_4']
    h = _gconv_pp_pallas([_ppt([cp0, cp1], c4['l1'], 16, 32)], c4['l1']['b'],
                         ei['pp3'], ea['pp3'], n3, 32)
    for lname in ('l2', 'l3', 'l4'):
        h = _gconv_pp_pallas([_ppt([h], c4[lname], 32, 32)], c4[lname]['b'],
                             ei['pp3'], ea['pp3'], n3, 32)
    u0, u1 = _edge_agg(h, ei['unpool32'], ea['unpool32'], n2)
    s4, q4 = _dense(n2, 32, [([u0, u1], None, False, None, None)],
                    stats=True, want_out=False)
    x4n = _dense(n2, 32, [([u0, u1], None, False, (s4, q4, n2), None)])

    def _cat2(lw, c_a, ci_pad_a, ci_pad_b, co_pad):
        return (_pad2(lw[:c_a], ci_pad_a, co_pad),
                _pad2(lw[c_a:], ci_pad_b, co_pad))

    c7 = p['convN_7']
    wr_a, wr_b = _cat2(c7['l1']['Wr'], 24, 32, 16, 32)
    wn_a, wn_b = _cat2(c7['l1']['Wn'], 24, 32, 16, 32)
    h = _gconv_pp_pallas([([x4n], wr_a, wn_a), ([x3n], wr_b, wn_b)],
                         c7['l1']['b'], ei['pp2'], ea['pp2'], n2, 32)
    h = _gconv_pp_pallas([_ppt([h], c7['l2'], 32, 32)], c7['l2']['b'],
                         ei['pp2'], ea['pp2'], n2, 32)
    u0, u1 = _edge_agg(h, ei['unpool21'], ea['unpool21'], n1)
    s7, q7 = _dense(n1, 32, [([u0, u1], None, False, None, None)],
                    stats=True, want_out=False)
    x7n = _dense(n1, 32, [([u0, u1], None, False, (s7, q7, n1), None)])

    c8 = p['convN_8']
    wr_a, wr_b = _cat2(c8['l1']['Wr'], 24, 32, 16, 16)
    wn_a, wn_b = _cat2(c8['l1']['Wn'], 24, 32, 16, 16)
    h = _gconv_pp_pallas([([x7n], wr_a, wn_a), ([x2n], wr_b, wn_b)],
                         c8['l1']['b'], ei['pp1'], ea['pp1'], n1, 16)
    h = _gconv_pp_pallas([_ppt([h], c8['l2'], 16, 16)], c8['l2']['b'],
                         ei['pp1'], ea['pp1'], n1, 16)
    u0, u1 = _edge_agg(h, ei['unpool10'], ea['unpool10'], n0)
    s8, q8 = _dense(n0, 16, [([u0, u1], None, False, None, None)],
                    stats=True, want_out=False)
    x8n = _dense(n0, 16, [([u0, u1], None, False, (s8, q8, n0), None)])

    c9 = p['convN_9']
    wr_a, wr_b = _cat2(c9['l1']['Wr'], 12, 16, 16, 16)
    wn_a, wn_b = _cat2(c9['l1']['Wn'], 12, 16, 16, 16)
    h = _gconv_pp_pallas([([x8n], wr_a, wn_a), ([x1n], wr_b, wn_b)],
                         c9['l1']['b'], ei['pp0'], ea['pp0'], n0, 16)
    xwpc = _dense(n0, 16, [([h], None, False, None, _pad2(c9['W_pc'], 16, 16))])
    g0, g1 = _edge_agg(xwpc, ei['pc'], ea['pc'], nc)

    fin = p['final']
    out = _dense(nc, 1, [([g0, g1], _padb(c9['b_pc'], 16), True, None,
                          _pad2(fin['W'], 16, 1))],
                 bias=_padb(fin['b'], 1))
    return out


# speedup vs baseline: 2.6137x; 1.0416x over previous
import functools

import jax
import jax.numpy as jnp
from jax import lax
from jax.experimental import pallas as pl
from jax.experimental.pallas import tpu as pltpu
from jax.experimental.pallas import tpu_sc as plsc

F32 = jnp.float32
I32 = jnp.int32

NCORES = 2
NSUB = 16
NTILES = NCORES * NSUB
CHS = 512
SUB = 128
EUNIT = NTILES * CHS
EPS = 1e-5


def _ceil_to(x, m):
    return (x + m - 1) // m * m


def _lane_take(v, idx):
    return lax.gather(
        v, idx[:, None],
        dimension_numbers=lax.GatherDimensionNumbers(
            offset_dims=(), collapsed_slice_dims=(0,), start_index_map=(0,)),
        slice_sizes=(1,), mode=lax.GatherScatterMode.PROMISE_IN_BOUNDS)


@functools.lru_cache(maxsize=None)
def _make_edge_agg(n_src, n_rows, k, e_pad, pack2):
    n_rows_pad = _ceil_to(n_rows, 512)
    rpt = n_rows_pad // NSUB
    ept = e_pad // NTILES
    e_half = e_pad // NCORES
    chunks = ept // CHS

    mesh = plsc.VectorSubcoreMesh(
        core_axis_name="c", subcore_axis_name="s",
        num_cores=NCORES, num_subcores=NSUB)

    def body(tbl, ei, ea, zeros_hbm, out, acc, srcb, dstb, eab, dloc,
             rows, sem, sem2):
        c = lax.axis_index("c")
        s = lax.axis_index("s")
        pltpu.sync_copy(zeros_hbm.at[pl.ds(s * rpt, rpt)],
                        acc.at[pl.ds(s * rpt, rpt)])
        plsc.subcore_barrier()

        base = c * e_half + s * ept
        lanes = lax.broadcasted_iota(I32, (16,), 0)

        def chunk_body(ch, carry):
            eb = base + ch * CHS
            c1 = pltpu.async_copy(ei.at[0, pl.ds(eb, CHS)], srcb, sem)
            c2 = pltpu.async_copy(ei.at[1, pl.ds(eb, CHS)], dstb, sem)
            c3 = pltpu.async_copy(ea.at[pl.ds(eb, CHS)], eab, sem)
            c1.wait(); c2.wait(); c3.wait()
            for sub in range(CHS // SUB):
                g = pltpu.async_copy(
                    tbl.at[srcb.at[pl.ds(sub * SUB, SUB)]], rows, sem2)
                g.wait()
                for grp in range(SUB // 16):
                    off = sub * SUB + grp * 16
                    ea_v = eab[pl.ds(off, 16)]
                    dstv = dstb[pl.ds(off, 16)]
                    if pack2:
                        dloc[pl.ds(grp * 16, 16)] = \
                            lax.shift_right_logical(dstv, 1)
                        par = dstv & 1
                    else:
                        dloc[pl.ds(grp * 16, 16)] = dstv
                    for e in range(16):
                        le = grp * 16 + e
                        idx_e = jnp.full((16,), e, I32)
                        scale = _lane_take(ea_v, idx_e)
                        if pack2:
                            pb = _lane_take(par, idx_e)
                            ridx = (lanes + lax.shift_left(pb, 3)) & 15
                        for h in range(k):
                            v = rows[le, h] * scale
                            if pack2:
                                v = _lane_take(v, ridx)
                            rows[le, h] = v
                pltpu.sync_copy(rows, acc.at[dloc], add=True)
            return carry

        lax.fori_loop(0, chunks, chunk_body, 0)
        plsc.subcore_barrier()
        pltpu.sync_copy(acc.at[pl.ds(s * rpt, rpt)],
                        out.at[c, pl.ds(s * rpt, rpt)])

    return pl.kernel(
        body,
        out_type=jax.ShapeDtypeStruct((NCORES, n_rows_pad, k, 16), F32),
        mesh=mesh,
        scratch_types=[
            pltpu.VMEM_SHARED((n_rows_pad, k, 16), F32),
            pltpu.VMEM((CHS,), I32),
            pltpu.VMEM((CHS,), I32),
            pltpu.VMEM((CHS,), F32),
            pltpu.VMEM((SUB,), I32),
            pltpu.VMEM((SUB, k, 16), F32),
            pltpu.SemaphoreType.DMA,
            pltpu.SemaphoreType.DMA,
        ],
        compiler_params=pltpu.CompilerParams(use_tc_tiling_on_sc=False),
    )


def _edge_agg(table, ei, ea, n_dst, pack2=False):
    nr, k, _ = table.shape
    e = ei.shape[1]
    e_pad = _ceil_to(e, EUNIT)
    if e_pad != e:
        ei = jnp.pad(ei, ((0, 0), (0, e_pad - e)))
        ea = jnp.pad(ea, (0, e_pad - e))
    n_rows = n_dst // 2 if pack2 else n_dst
    n_rows_pad = _ceil_to(n_rows, 512)
    zeros_hbm = jnp.zeros((n_rows_pad, k, 16), F32)
    fn = _make_edge_agg(nr, n_rows, k, e_pad, pack2)
    return fn(table, ei, ea, zeros_hbm)


def _dense(n, co, terms, bias=None, relu=False, stats=False, want_out=True,
           out3=False):
    np_ = _ceil_to(n, 8)
    r = 1000 if np_ % 1000 == 0 else np_
    grid = np_ // r
    has_pad_rows = np_ != n

    cfg = []
    operands = []
    in_specs = []

    def add_x(a):
        if a.ndim == 2:
            ci = a.shape[1]
            operands.append(a)
            in_specs.append(pl.BlockSpec((r, ci), lambda i: (i, 0)))
            return ci, 2
        if a.ndim == 3:
            k = a.shape[1]
            operands.append(a)
            in_specs.append(pl.BlockSpec((r, k, 16), lambda i: (i, 0, 0)))
            return k * 16, 3
        k = a.shape[2]
        operands.append(a)
        in_specs.append(pl.BlockSpec((2, r, k, 16), lambda i: (0, i, 0, 0)))
        return k * 16, 4

    def add_full(a, ci):
        operands.append(a)
        in_specs.append(pl.BlockSpec((1, ci), lambda i: (0, 0)))

    for (xs, tb, trelu, norm, w) in terms:
        kinds = []
        ci = None
        for x in xs:
            ci, kind = add_x(x)
            kinds.append(kind)
        if tb is not None:
            add_full(tb, ci)
        cnt = None
        nmode = None
        if norm is not None:
            ssum, svar, cnt, nmode = norm
            add_full(ssum, ssum.shape[1])
            if svar is not None:
                add_full(svar, svar.shape[1])
        if w is not None:
            operands.append(w)
            in_specs.append(pl.BlockSpec(w.shape, lambda i: (0, 0)))
        cfg.append((tuple(kinds), ci, tb is not None, trelu, nmode,
                    w is not None, float(cnt) if cnt else 0.0))
    if bias is not None:
        add_full(bias, co)

    out_shapes = []
    out_specs = []
    if want_out:
        if out3:
            ko = co // 16
            out_shapes.append(jax.ShapeDtypeStruct((np_, ko, 16), F32))
            out_specs.append(pl.BlockSpec((r, ko, 16), lambda i: (i, 0, 0)))
        else:
            out_shapes.append(jax.ShapeDtypeStruct((np_, co), F32))
            out_specs.append(pl.BlockSpec((r, co), lambda i: (i, 0)))
    if stats:
        for _ in range(2):
            out_shapes.append(jax.ShapeDtypeStruct((1, co), F32))
            out_specs.append(pl.BlockSpec((1, co), lambda i: (0, 0)))

    has_bias = bias is not None

    def body(*refs):
        it = iter(refs)
        acc = None
        for (kinds, ci, has_tb, trelu, nmode, has_w, cnt) in cfg:
            xb = None
            for kind in kinds:
                x = next(it)[...]
                if kind == 3:
                    x = x.reshape(r, ci)
                elif kind == 4:
                    x = (x[0] + x[1]).reshape(r, ci)
                xb = x if xb is None else xb + x
            if has_tb:
                xb = xb + next(it)[...]
            if trelu:
                xb = jnp.maximum(xb, 0.0)
            def _fold(a):
                hw = a.shape[1] // 2
                return a[:, :hw] + a[:, hw:]

            if nmode == 'var':
                m = next(it)[...] / cnt
                v = next(it)[...] / cnt
                xb = (xb - m) / jnp.sqrt(v + EPS)
            elif nmode == 'center':
                xb = xb - next(it)[...] / cnt
            elif nmode == 'fold2var':
                m = _fold(next(it)[...]) / cnt
                v = _fold(next(it)[...]) / cnt
                rs = 1.0 / jnp.sqrt(v + EPS)
                xb = (xb - jnp.concatenate([m, m], axis=1)) \
                    * jnp.concatenate([rs, rs], axis=1)
            elif nmode == 'center2':
                m = _fold(next(it)[...]) / cnt
                xb = xb - jnp.concatenate([m, m], axis=1)
            if has_w:
                xb = jnp.dot(xb.astype(jnp.bfloat16),
                             next(it)[...].astype(jnp.bfloat16),
                             preferred_element_type=F32)
            acc = xb if acc is None else acc + xb
        if has_bias:
            acc = acc + next(it)[...]
        if relu:
            acc = jnp.maximum(acc, 0.0)
        if want_out:
            oref = next(it)
            if out3:
                oref[...] = acc.reshape(r, co // 16, 16)
            else:
                oref[...] = acc
        if stats:
            sr = next(it)
            qr = next(it)
            i = pl.program_id(0)
            a = acc
            if has_pad_rows:
                row = lax.broadcasted_iota(I32, (r, 1), 0) + i * r
                a = jnp.where(row < n, acc, 0.0)
            bs = jnp.sum(a, axis=0, keepdims=True)
            bq = jnp.sum(a * a, axis=0, keepdims=True)

            @pl.when(i == 0)
            def _():
                sr[...] = bs
                qr[...] = bq

            @pl.when(i > 0)
            def _():
                sr[...] = sr[...] + bs
                qr[...] = qr[...] + bq

    res = pl.pallas_call(
        body,
        grid=(grid,),
        in_specs=in_specs,
        out_specs=out_specs if len(out_specs) > 1 else out_specs[0],
        out_shape=out_shapes if len(out_shapes) > 1 else out_shapes[0],
    )(*operands)
    return res


def _pad2(w, ri, co):
    out = jnp.zeros((ri, co), F32)
    return out.at[:w.shape[0], :w.shape[1]].set(w)


def _padb(b, co):
    return jnp.zeros((1, co), F32).at[0, :b.shape[0]].set(b)


def _gconv_pp_pallas(x_terms, b, ei, ea, n, co_pad, stats=False):
    wn_terms = [(xs, None, False, None, wn_i) for (xs, _, wn_i) in x_terms]
    wr_terms = [(xs, None, False, None, wr_i) for (xs, wr_i, _) in x_terms]
    xwn = _dense(n, co_pad, wn_terms, out3=True)
    agg = _edge_agg(xwn, ei, ea, n)
    out_terms = wr_terms + [([agg], None, False, None, None)]
    return _dense(n, co_pad, out_terms, bias=_padb(b, co_pad), relu=True,
                  stats=stats, out3=True)


def kernel(xCellCenters, xFace, params, edge_attrs, edge_indices):
    p = params
    ea = edge_attrs
    ei = edge_indices
    nc = xCellCenters.shape[1]
    nf = xFace.shape[1]
    n0 = 100000
    n1 = 25000
    n2 = 6250
    n3 = 1600

    xcc = xCellCenters[0]
    xfp = jnp.reshape(xFace[0], (nf // 2, 8))

    def _centered_var(n, c, xs, ssum, cnt=None, mode='center'):
        res = _dense(n, c, [(xs, None, False,
                             (ssum, None, cnt or n, mode), None)],
                     stats=True, want_out=False)
        return res[1]

    scc, _ = _dense(nc, 2, [([xcc], None, False, None, None)],
                    stats=True, want_out=False)
    vcc = _centered_var(nc, 2, [xcc], scc)
    sf, _ = _dense(nf // 2, 8, [([xfp], None, False, None, None)],
                   stats=True, want_out=False)
    vf = _centered_var(nf // 2, 8, [xfp], sf, cnt=nf, mode='center2')

    c1 = p['convN_1']
    xwcf = _dense(nc, 16, [([xcc], None, False, (scc, vcc, nc, 'var'),
                            _pad2(c1['W_cf'], 2, 16))], out3=True)
    acf = _edge_agg(xwcf, ei['cf'], ea['cf'], nf, pack2=True)

    wfp = c1['W_fp']
    wtop = wfp[:8]
    wbot = wfp[8:]
    wA = jnp.zeros((16, 32), F32).at[:8, :12].set(wtop).at[8:16, 16:28].set(wtop)
    wB = jnp.zeros((8, 32), F32).at[:4, :12].set(wbot).at[4:8, 16:28].set(wbot)
    bA = jnp.zeros((1, 16), F32).at[0, :8].set(c1['b_cf']).at[0, 8:].set(c1['b_cf'])
    xwfp_packed = _dense(nf // 2, 32, [
        ([acf], bA, True, None, wA),
        ([xfp], None, False, (sf, vf, nf, 'fold2var'), wB),
    ])
    xwfp = jnp.reshape(xwfp_packed, (nf, 1, 16))
    afp = _edge_agg(xwfp, ei['fp'], ea['fp'], n0)
    h0 = _dense(n0, 16, [([afp], _padb(c1['b_fp'], 16), True, None, None)],
                out3=True)

    x1, s1, _ = _gconv_pp_pallas(
        [([h0], _pad2(c1['Wr_pp'], 16, 16), _pad2(c1['Wn_pp'], 16, 16))],
        c1['b_pp'], ei['pp0'], ea['pp0'], n0, 16, stats=True)
    v1 = _centered_var(n0, 16, [x1], s1)
    x1n = _dense(n0, 16, [([x1], None, False, (s1, v1, n0, 'var'), None)],
                 out3=True)

    def _ppt(xs, lw, ci_pad, co_pad):
        return (xs, _pad2(lw['Wr'], ci_pad, co_pad),
                _pad2(lw['Wn'], ci_pad, co_pad))

    ap = _edge_agg(x1n, ei['pool01'], ea['pool01'], n1)
    c2 = p['convN_2']
    h = _gconv_pp_pallas([_ppt([ap], c2['l1'], 16, 16)], c2['l1']['b'],
                         ei['pp1'], ea['pp1'], n1, 16)
    x2, s2, _ = _gconv_pp_pallas([_ppt([h], c2['l2'], 16, 16)], c2['l2']['b'],
                                 ei['pp1'], ea['pp1'], n1, 16, stats=True)
    v2 = _centered_var(n1, 16, [x2], s2)
    x2n = _dense(n1, 16, [([x2], None, False, (s2, v2, n1, 'var'), None)],
                 out3=True)

    bp = _edge_agg(x2n, ei['pool12'], ea['pool12'], n2)
    c3 = p['convN_3']
    h = _gconv_pp_pallas([_ppt([bp], c3['l1'], 16, 16)], c3['l1']['b'],
                         ei['pp2'], ea['pp2'], n2, 16)
    x3, s3, _ = _gconv_pp_pallas([_ppt([h], c3['l2'], 16, 16)], c3['l2']['b'],
                                 ei['pp2'], ea['pp2'], n2, 16, stats=True)
    v3 = _centered_var(n2, 16, [x3], s3)
    x3n = _dense(n2, 16, [([x3], None, False, (s3, v3, n2, 'var'), None)],
                 out3=True)

    cp = _edge_agg(x3n, ei['pool23'], ea['pool23'], n3)
    c4 = p['convN_4']
    h = _gconv_pp_pallas([_ppt([cp], c4['l1'], 16, 32)], c4['l1']['b'],
                         ei['pp3'], ea['pp3'], n3, 32)
    for lname in ('l2', 'l3', 'l4'):
        h = _gconv_pp_pallas([_ppt([h], c4[lname], 32, 32)], c4[lname]['b'],
                             ei['pp3'], ea['pp3'], n3, 32)
    up = _edge_agg(h, ei['unpool32'], ea['unpool32'], n2)
    s4, _ = _dense(n2, 32, [([up], None, False, None, None)],
                   stats=True, want_out=False)
    v4 = _centered_var(n2, 32, [up], s4)
    x4n = _dense(n2, 32, [([up], None, False, (s4, v4, n2, 'var'), None)],
                 out3=True)

    def _cat2(lw, c_a, ci_pad_a, ci_pad_b, co_pad):
        return (_pad2(lw[:c_a], ci_pad_a, co_pad),
                _pad2(lw[c_a:], ci_pad_b, co_pad))

    c7 = p['convN_7']
    wr_a, wr_b = _cat2(c7['l1']['Wr'], 24, 32, 16, 32)
    wn_a, wn_b = _cat2(c7['l1']['Wn'], 24, 32, 16, 32)
    h = _gconv_pp_pallas([([x4n], wr_a, wn_a), ([x3n], wr_b, wn_b)],
                         c7['l1']['b'], ei['pp2'], ea['pp2'], n2, 32)
    h = _gconv_pp_pallas([_ppt([h], c7['l2'], 32, 32)], c7['l2']['b'],
                         ei['pp2'], ea['pp2'], n2, 32)
    u7 = _edge_agg(h, ei['unpool21'], ea['unpool21'], n1)
    s7, _ = _dense(n1, 32, [([u7], None, False, None, None)],
                   stats=True, want_out=False)
    v7 = _centered_var(n1, 32, [u7], s7)
    x7n = _dense(n1, 32, [([u7], None, False, (s7, v7, n1, 'var'), None)],
                 out3=True)

    c8 = p['convN_8']
    wr_a, wr_b = _cat2(c8['l1']['Wr'], 24, 32, 16, 16)
    wn_a, wn_b = _cat2(c8['l1']['Wn'], 24, 32, 16, 16)
    h = _gconv_pp_pallas([([x7n], wr_a, wn_a), ([x2n], wr_b, wn_b)],
                         c8['l1']['b'], ei['pp1'], ea['pp1'], n1, 16)
    h = _gconv_pp_pallas([_ppt([h], c8['l2'], 16, 16)], c8['l2']['b'],
                         ei['pp1'], ea['pp1'], n1, 16)
    u8 = _edge_agg(h, ei['unpool10'], ea['unpool10'], n0)
    s8, _ = _dense(n0, 16, [([u8], None, False, None, None)],
                   stats=True, want_out=False)
    v8 = _centered_var(n0, 16, [u8], s8)
    x8n = _dense(n0, 16, [([u8], None, False, (s8, v8, n0, 'var'), None)],
                 out3=True)

    c9 = p['convN_9']
    wr_a, wr_b = _cat2(c9['l1']['Wr'], 12, 16, 16, 16)
    wn_a, wn_b = _cat2(c9['l1']['Wn'], 12, 16, 16, 16)
    h = _gconv_pp_pallas([([x8n], wr_a, wn_a), ([x1n], wr_b, wn_b)],
                         c9['l1']['b'], ei['pp0'], ea['pp0'], n0, 16)
    xwpc = _dense(n0, 16, [([h], None, False, None, _pad2(c9['W_pc'], 16, 16))],
                  out3=True)
    g = _edge_agg(xwpc, ei['pc'], ea['pc'], nc)

    fin = p['final']
    out = _dense(nc, 1, [([g], _padb(c9['b_pc'], 16), True, None,
                          _pad2(fin['W'], 16, 1))],
                 bias=_padb(fin['b'], 1))
    return out
